# Initial kernel scaffold; baseline (speedup 1.0000x reference)
import jax
import jax.numpy as jnp
from jax.experimental import pallas as pl


def kernel(queries, keys, values, clusters):
    raise NotImplementedError("write your pallas kernel here")


# speedup vs baseline: 6.2158x; 6.2158x over previous
import functools

import jax
import jax.numpy as jnp
from jax import lax
from jax.experimental import pallas as pl
from jax.experimental.pallas import tpu as pltpu
from jax.experimental.pallas import tpu_sc as plsc

B, N, D = 32, 8192, 64
C = 129
CP = 136
NC, NS = 2, 16
NW = NC * NS
CH = 128
SG = 1024
SUB = SG // CH
NSG = N // SG

_MESH = plsc.VectorSubcoreMesh(
    core_axis_name="c", subcore_axis_name="s", num_cores=NC, num_subcores=NS
)
_SC_PARAMS = pltpu.CompilerParams(use_tc_tiling_on_sc=False)


def _sc_reduce_body(q_hbm, k_hbm, v_hbm, idx_hbm, z64_hbm, z16_hbm, ones_hbm,
                    qs_hbm, ks_hbm, vs_hbm, cnt_hbm,
                    sq, sk, sv, scnt, idxv, buf, ones_v):
    s = lax.axis_index("s")
    b = s * NC + lax.axis_index("c")
    reg = pl.ds(s * CP, CP)
    pltpu.sync_copy(z64_hbm, sq.at[reg, :])
    pltpu.sync_copy(z64_hbm, sk.at[reg, :])
    pltpu.sync_copy(z64_hbm, sv.at[reg, :])
    pltpu.sync_copy(z16_hbm, scnt.at[reg, :])
    pltpu.sync_copy(ones_hbm, ones_v)
    pltpu.sync_copy(idx_hbm.at[b], idxv)

    def tensor_pass(x_hbm, acc):
        def chunk(g, carry):
            pltpu.sync_copy(x_hbm.at[b, pl.ds(g * SG, SG), :], buf)
            for m in range(SUB):
                pltpu.sync_copy(buf.at[pl.ds(m * CH, CH), :],
                                acc.at[idxv.at[g * SUB + m]], add=True)
            return carry
        lax.fori_loop(0, NSG, chunk, 0)

    tensor_pass(q_hbm, sq)
    tensor_pass(k_hbm, sk)
    tensor_pass(v_hbm, sv)

    def count_chunk(j, carry):
        pltpu.sync_copy(ones_v, scnt.at[idxv.at[j]], add=True)
        return carry
    lax.fori_loop(0, N // CH, count_chunk, 0)

    pltpu.sync_copy(sq.at[reg, :], qs_hbm.at[b])
    pltpu.sync_copy(sk.at[reg, :], ks_hbm.at[b])
    pltpu.sync_copy(sv.at[reg, :], vs_hbm.at[b])
    pltpu.sync_copy(scnt.at[reg, :], cnt_hbm.at[b])


_sc_reduce = pl.kernel(
    _sc_reduce_body,
    out_type=[
        jax.ShapeDtypeStruct((B, CP, D), jnp.float32),
        jax.ShapeDtypeStruct((B, CP, D), jnp.float32),
        jax.ShapeDtypeStruct((B, CP, D), jnp.float32),
        jax.ShapeDtypeStruct((B, CP, 16), jnp.float32),
    ],
    mesh=_MESH,
    scratch_types=[
        pltpu.VMEM_SHARED((NS * CP, D), jnp.float32),
        pltpu.VMEM_SHARED((NS * CP, D), jnp.float32),
        pltpu.VMEM_SHARED((NS * CP, D), jnp.float32),
        pltpu.VMEM_SHARED((NS * CP, 16), jnp.float32),
        pltpu.VMEM((N // CH, CH), jnp.int32),
        pltpu.VMEM((SG, D), jnp.float32),
        pltpu.VMEM((CH, 16), jnp.float32),
    ],
    compiler_params=_SC_PARAMS,
)


def _tc_attn_body(q_ref, k_ref, v_ref, c_ref, vout_ref, a0_ref):
    counts_col = c_ref[0][:, 0:1]
    counts_row = jnp.reshape(c_ref[0][:, 0], (1, CP))
    inv = 1.0 / jnp.maximum(counts_col, 1.0)
    qc = q_ref[0] * inv
    kc = k_ref[0] * inv
    vc = v_ref[0] * inv
    qk = lax.dot_general(qc, kc, (((1,), (1,)), ((), ())),
                         preferred_element_type=jnp.float32)
    m = jnp.max(qk, axis=1, keepdims=True)
    w = jnp.exp(qk - m) * counts_row
    a = w / jnp.sum(w, axis=1, keepdims=True)
    vout_ref[0] = lax.dot_general(a, vc, (((1,), (0,)), ((), ())),
                                  preferred_element_type=jnp.float32)
    a0_ref[0] = a[:, 0:1]


_tc_attn = pl.pallas_call(
    _tc_attn_body,
    grid=(B,),
    in_specs=[
        pl.BlockSpec((1, CP, D), lambda b: (b, 0, 0)),
        pl.BlockSpec((1, CP, D), lambda b: (b, 0, 0)),
        pl.BlockSpec((1, CP, D), lambda b: (b, 0, 0)),
        pl.BlockSpec((1, CP, 16), lambda b: (b, 0, 0)),
    ],
    out_specs=[
        pl.BlockSpec((1, CP, D), lambda b: (b, 0, 0)),
        pl.BlockSpec((1, CP, 1), lambda b: (b, 0, 0)),
    ],
    out_shape=[
        jax.ShapeDtypeStruct((B, CP, D), jnp.float32),
        jax.ShapeDtypeStruct((B, CP, 1), jnp.float32),
    ],
)


def _sc_gather_body(vout_hbm, idx_hbm, out_hbm, stab, idxv, buf):
    s = lax.axis_index("s")
    b = s * NC + lax.axis_index("c")
    pltpu.sync_copy(vout_hbm.at[b], stab.at[pl.ds(s * CP, CP), :])
    pltpu.sync_copy(idx_hbm.at[b], idxv)

    def chunk(g, carry):
        for m in range(SUB):
            pltpu.sync_copy(stab.at[idxv.at[g * SUB + m]],
                            buf.at[pl.ds(m * CH, CH), :])
        pltpu.sync_copy(buf, out_hbm.at[b, pl.ds(g * SG, SG), :])
        return carry
    lax.fori_loop(0, NSG, chunk, 0)


_sc_gather = pl.kernel(
    _sc_gather_body,
    out_type=jax.ShapeDtypeStruct((B, N, D), jnp.float32),
    mesh=_MESH,
    scratch_types=[
        pltpu.VMEM_SHARED((NS * CP, D), jnp.float32),
        pltpu.VMEM((N // CH, CH), jnp.int32),
        pltpu.VMEM((SG, D), jnp.float32),
    ],
    compiler_params=_SC_PARAMS,
)


@jax.jit
def _run(queries, keys, values, clusters):
    offs = (jnp.arange(B, dtype=jnp.int32) // NC * CP)[:, None]
    idx3 = (clusters + offs).reshape(B, N // CH, CH)
    z64 = jnp.zeros((CP, D), jnp.float32)
    z16 = jnp.zeros((CP, 16), jnp.float32)
    ones = jnp.ones((CH, 16), jnp.float32)
    qs, ks, vs, cnt = _sc_reduce(queries, keys, values, idx3, z64, z16, ones)
    vout, a0 = _tc_attn(qs, ks, vs, cnt)
    out = _sc_gather(vout, idx3)
    return out, a0[:, :C, 0]


def kernel(queries, keys, values, clusters):
    return _run(queries, keys, values, clusters.astype(jnp.int32))
